# baseline (speedup 1.0000x reference)
import functools

import jax
import jax.numpy as jnp
from jax import lax
from jax.experimental import pallas as pl
from jax.experimental.pallas import tpu as pltpu
from jax.experimental.pallas import tpu_sc as plsc

N_ROWS = 16384
N_ATTR = 26
D = 64
B = N_ROWS * N_ATTR

NUM_CORES = 2
NUM_SUBCORES = 16
NW = NUM_CORES * NUM_SUBCORES

CHUNK = 128
CHUNKS_TOTAL = B // CHUNK
CH_PER_W = CHUNKS_TOTAL // NW


@functools.partial(
    pl.kernel,
    mesh=plsc.VectorSubcoreMesh(core_axis_name="c", subcore_axis_name="s"),
    out_type=jax.ShapeDtypeStruct((B, D), jnp.float32),
    compiler_params=pltpu.CompilerParams(use_tc_tiling_on_sc=False),
    scratch_types=[
        pltpu.VMEM((CH_PER_W, CHUNK), jnp.int32),
        pltpu.VMEM((CHUNK, D), jnp.float32),
        pltpu.SemaphoreType.DMA,
    ],
)
def _gather_kernel(idx_hbm, table_hbm, out_hbm, idx_v, rows_v, sem):
    wid = lax.axis_index("s") * NUM_CORES + lax.axis_index("c")
    chunk_base = wid * CH_PER_W

    pltpu.sync_copy(idx_hbm.at[pl.ds(chunk_base, CH_PER_W)], idx_v)

    def step(j, carry):
        pltpu.async_copy(table_hbm.at[idx_v.at[j]], rows_v, sem).wait()
        pltpu.sync_copy(
            rows_v, out_hbm.at[pl.ds((chunk_base + j) * CHUNK, CHUNK)]
        )
        return carry

    lax.fori_loop(0, CH_PER_W, step, 0)


def kernel(x, table):
    idx = x.reshape(CHUNKS_TOTAL, CHUNK).astype(jnp.int32)
    out = _gather_kernel(idx, table)
    return out.reshape(N_ROWS, N_ATTR, D)


# speedup vs baseline: 1.0763x; 1.0763x over previous
import functools

import jax
import jax.numpy as jnp
from jax import lax
from jax.experimental import pallas as pl
from jax.experimental.pallas import tpu as pltpu
from jax.experimental.pallas import tpu_sc as plsc

N_ROWS = 16384
N_ATTR = 26
D = 64
B = N_ROWS * N_ATTR

NUM_CORES = 2
NUM_SUBCORES = 16
NW = NUM_CORES * NUM_SUBCORES

CHUNK = 128
CHUNKS_TOTAL = B // CHUNK
CH_PER_W = CHUNKS_TOTAL // NW
NBUF = 8
ROUNDS = CH_PER_W // NBUF


@functools.partial(
    pl.kernel,
    mesh=plsc.VectorSubcoreMesh(core_axis_name="c", subcore_axis_name="s"),
    out_type=jax.ShapeDtypeStruct((B, D), jnp.float32),
    compiler_params=pltpu.CompilerParams(use_tc_tiling_on_sc=False),
    scratch_types=(
        [pltpu.VMEM((CH_PER_W, CHUNK), jnp.int32)]
        + [pltpu.VMEM((CHUNK, D), jnp.float32) for _ in range(NBUF)]
        + [pltpu.SemaphoreType.DMA for _ in range(2 * NBUF)]
    ),
)
def _gather_kernel(idx_hbm, table_hbm, out_hbm, idx_v, *bufs_and_sems):
    rows = bufs_and_sems[:NBUF]
    gsem = bufs_and_sems[NBUF : 2 * NBUF]
    ssem = bufs_and_sems[2 * NBUF : 3 * NBUF]

    wid = lax.axis_index("s") * NUM_CORES + lax.axis_index("c")
    chunk_base = wid * CH_PER_W

    pltpu.sync_copy(idx_hbm.at[pl.ds(chunk_base, CH_PER_W)], idx_v)

    def gather_start(j, b):
        pltpu.make_async_copy(
            table_hbm.at[idx_v.at[j]], rows[b], gsem[b]
        ).start()

    def gather_wait(j, b):
        pltpu.make_async_copy(
            table_hbm.at[idx_v.at[j]], rows[b], gsem[b]
        ).wait()

    def store_start(j, b):
        pltpu.make_async_copy(
            rows[b], out_hbm.at[pl.ds((chunk_base + j) * CHUNK, CHUNK)], ssem[b]
        ).start()

    def store_wait(j, b):
        pltpu.make_async_copy(
            rows[b], out_hbm.at[pl.ds((chunk_base + j) * CHUNK, CHUNK)], ssem[b]
        ).wait()

    for b in range(NBUF):
        gather_start(b, b)

    def round_body(g, carry):
        for b in range(NBUF):
            j = g * NBUF + b
            gather_wait(j, b)
            store_start(j, b)
        for b in range(NBUF):
            j = g * NBUF + b
            store_wait(j, b)
            gather_start(j + NBUF, b)
        return carry

    lax.fori_loop(0, ROUNDS - 1, round_body, 0)

    g = ROUNDS - 1
    for b in range(NBUF):
        j = g * NBUF + b
        gather_wait(j, b)
        store_start(j, b)
    for b in range(NBUF):
        j = g * NBUF + b
        store_wait(j, b)


def kernel(x, table):
    idx = x.reshape(CHUNKS_TOTAL, CHUNK).astype(jnp.int32)
    out = _gather_kernel(idx, table)
    return out.reshape(N_ROWS, N_ATTR, D)
